# baseline (speedup 1.0000x reference)
import functools

import jax
import jax.numpy as jnp
from jax import lax
from jax.experimental import pallas as pl
from jax.experimental.pallas import tpu as pltpu
from jax.experimental.pallas import tpu_sc as plsc

BATCH = 16384
NUM_FIELDS = 26
EMBED_DIM = 32
VOCAB = 1000000
TOTAL = BATCH * NUM_FIELDS
NC = 2
NS = 16
NW = NC * NS

_mesh = plsc.VectorSubcoreMesh(core_axis_name="c", subcore_axis_name="s")


V_PER_W = 31232
BLK = 1024
N_FULL = V_PER_W // BLK
V_MAIN = NW * V_PER_W
V_REM = 999936 - V_MAIN
V_TAIL = VOCAB - 999936


@functools.partial(
    pl.kernel,
    out_type=jax.ShapeDtypeStruct((VOCAB // 4, 128), jnp.float32),
    mesh=_mesh,
    scratch_types=[
        pltpu.VMEM((2, 32, BLK), jnp.float32),
        pltpu.VMEM((BLK // 4, 128), jnp.float32),
        [pltpu.SemaphoreType.DMA] * 2,
    ],
    compiler_params=pltpu.CompilerParams(
        use_tc_tiling_on_sc=True, needs_layout_passes=False
    ),
)
def _table_transpose(tt_hbm, tail_hbm, tp_hbm, src, dst, gsems):
    wid = lax.axis_index("s") * NC + lax.axis_index("c")
    v0w = wid * V_PER_W

    def issue_reads(v0, width, buf):
        v0 = pl.multiple_of(v0, 128)
        return [
            pltpu.async_copy(
                tt_hbm.at[pl.ds(8 * tr, 8), pl.ds(v0, width)],
                src.at[buf, pl.ds(8 * tr, 8), pl.ds(0, width)],
                gsems[buf],
            )
            for tr in range(4)
        ]

    d_lo = lax.iota(jnp.int32, 16)
    d_hi = d_lo + 16

    def transpose_block(buf, width):
        src_b = src.at[buf]

        def body(r, _):
            for jj in range(4):
                vsp = jnp.full((16,), 4 * r + jj, jnp.int32)
                g0 = plsc.load_gather(src_b, [d_lo, vsp])
                g1 = plsc.load_gather(src_b, [d_hi, vsp])
                dst[r, pl.ds(32 * jj, 16)] = g0
                dst[r, pl.ds(32 * jj + 16, 16)] = g1
            return 0

        lax.fori_loop(0, width // 4, body, 0)

    widths = [BLK] * N_FULL + [V_PER_W - N_FULL * BLK]
    v0s = [v0w + k * BLK for k in range(len(widths))]

    reads = [issue_reads(v0s[0], widths[0], 0)]
    for k in range(len(widths)):
        if k + 1 < len(widths):
            reads.append(issue_reads(v0s[k + 1], widths[k + 1], (k + 1) % 2))
        for r_ in reads[k]:
            r_.wait()
        transpose_block(k % 2, widths[k])
        pltpu.sync_copy(
            dst.at[pl.ds(0, widths[k] // 4)],
            tp_hbm.at[pl.ds(pl.multiple_of(v0s[k] // 4, 8), widths[k] // 4)],
        )

    @pl.when(wid == 0)
    def _():
        rem = issue_reads(V_MAIN, V_REM, 0)
        for r_ in rem:
            r_.wait()
        transpose_block(0, V_REM)
        pltpu.sync_copy(
            dst.at[pl.ds(0, V_REM // 4)],
            tp_hbm.at[pl.ds(V_MAIN // 4, V_REM // 4)],
        )

    @pl.when(wid == 1)
    def _():
        pltpu.sync_copy(tail_hbm, dst.at[pl.ds(0, 16)])
        pltpu.sync_copy(
            dst.at[pl.ds(0, 16)], tp_hbm.at[pl.ds(999936 // 4, 16)]
        )


B_PER_W = TOTAL // NW
CHUNK = 832
N_CHUNKS = B_PER_W // CHUNK
NBUF = 4


@functools.partial(
    pl.kernel,
    out_type=jax.ShapeDtypeStruct((TOTAL, EMBED_DIM), jnp.float32),
    mesh=_mesh,
    scratch_types=[
        pltpu.VMEM((B_PER_W,), jnp.int32),
        pltpu.VMEM((NBUF, CHUNK, EMBED_DIM), jnp.float32),
        [pltpu.SemaphoreType.DMA] * NBUF,
        [pltpu.SemaphoreType.DMA] * NBUF,
    ],
    compiler_params=pltpu.CompilerParams(use_tc_tiling_on_sc=False),
)
def _embed_gather(idx_hbm, table_hbm, out_hbm, idx_v, rows_v, gsems, ssems):
    wid = lax.axis_index("s") * NC + lax.axis_index("c")
    base = wid * B_PER_W
    pltpu.sync_copy(idx_hbm.at[pl.ds(base, B_PER_W)], idx_v)

    def start_gather(c):
        b = c % NBUF
        return pltpu.async_copy(
            table_hbm.at[idx_v.at[pl.ds(c * CHUNK, CHUNK)]],
            rows_v.at[b],
            gsems[b],
        )

    def start_store(c):
        b = c % NBUF
        return pltpu.async_copy(
            rows_v.at[b],
            out_hbm.at[pl.ds(base + c * CHUNK, CHUNK)],
            ssems[b],
        )

    gathers = [None] * N_CHUNKS
    stores = [None] * N_CHUNKS
    for c in range(min(NBUF - 1, N_CHUNKS)):
        gathers[c] = start_gather(c)
    for c in range(N_CHUNKS):
        if c > 0:
            stores[c - 1].wait()
        g = c + NBUF - 1
        if g < N_CHUNKS:
            gathers[g] = start_gather(g)
        gathers[c].wait()
        stores[c] = start_store(c)
    stores[N_CHUNKS - 1].wait()


def kernel(x, table):
    flat = x.reshape(TOTAL).astype(jnp.int32)
    tt = table.T
    tail16 = lax.slice(table, (999936, 0), (VOCAB, EMBED_DIM)).reshape(16, 128)
    tp = _table_transpose(tt, tail16)
    tlin = tp.reshape(VOCAB, EMBED_DIM)
    out = _embed_gather(flat, tlin)
    return out.reshape(BATCH, NUM_FIELDS, EMBED_DIM)


# speedup vs baseline: 1.0515x; 1.0515x over previous
import functools

import jax
import jax.numpy as jnp
from jax import lax
from jax.experimental import pallas as pl
from jax.experimental.pallas import tpu as pltpu
from jax.experimental.pallas import tpu_sc as plsc

BATCH = 16384
NUM_FIELDS = 26
EMBED_DIM = 32
VOCAB = 1000000
TOTAL = BATCH * NUM_FIELDS
NC = 2
NS = 16
NW = NC * NS

_mesh = plsc.VectorSubcoreMesh(core_axis_name="c", subcore_axis_name="s")


V_PER_W = 31232
BLK = 1024
N_FULL = V_PER_W // BLK
V_MAIN = NW * V_PER_W
V_REM = 999936 - V_MAIN
V_TAIL = VOCAB - 999936


@functools.partial(
    pl.kernel,
    out_type=jax.ShapeDtypeStruct((VOCAB // 4, 128), jnp.float32),
    mesh=_mesh,
    scratch_types=[
        pltpu.VMEM((2, 32, BLK), jnp.float32),
        pltpu.VMEM((BLK // 4, 128), jnp.float32),
        [pltpu.SemaphoreType.DMA] * 2,
    ],
    compiler_params=pltpu.CompilerParams(
        use_tc_tiling_on_sc=True, needs_layout_passes=False
    ),
)
def _table_transpose(tt_hbm, tail_hbm, tp_hbm, src, dst, gsems):
    wid = lax.axis_index("s") * NC + lax.axis_index("c")
    v0w = wid * V_PER_W

    def issue_reads(v0, width, buf):
        v0 = pl.multiple_of(v0, 128)
        return [
            pltpu.async_copy(
                tt_hbm.at[pl.ds(8 * tr, 8), pl.ds(v0, width)],
                src.at[buf, pl.ds(8 * tr, 8), pl.ds(0, width)],
                gsems[buf],
            )
            for tr in range(4)
        ]

    iota = lax.iota(jnp.int32, 16)
    row_pat = iota // 4
    lane_pat = (iota % 4) * 32

    def transpose_block(buf, width):
        src_b = src.at[buf]

        def body(k, _):
            idx_r = row_pat + 4 * k
            for d in range(32):
                vreg = src_b[d, pl.ds(16 * k, 16)]
                plsc.store_scatter(dst, [idx_r, lane_pat + d], vreg)
            return 0

        lax.fori_loop(0, width // 16, body, 0)

    widths = [BLK] * N_FULL + [V_PER_W - N_FULL * BLK]
    v0s = [v0w + k * BLK for k in range(len(widths))]

    reads = [issue_reads(v0s[0], widths[0], 0)]
    for k in range(len(widths)):
        if k + 1 < len(widths):
            reads.append(issue_reads(v0s[k + 1], widths[k + 1], (k + 1) % 2))
        for r_ in reads[k]:
            r_.wait()
        transpose_block(k % 2, widths[k])
        pltpu.sync_copy(
            dst.at[pl.ds(0, widths[k] // 4)],
            tp_hbm.at[pl.ds(pl.multiple_of(v0s[k] // 4, 8), widths[k] // 4)],
        )

    @pl.when(wid == 0)
    def _():
        rem = issue_reads(V_MAIN, V_REM, 0)
        for r_ in rem:
            r_.wait()
        transpose_block(0, V_REM)
        pltpu.sync_copy(
            dst.at[pl.ds(0, V_REM // 4)],
            tp_hbm.at[pl.ds(V_MAIN // 4, V_REM // 4)],
        )

    @pl.when(wid == 1)
    def _():
        pltpu.sync_copy(tail_hbm, dst.at[pl.ds(0, 16)])
        pltpu.sync_copy(
            dst.at[pl.ds(0, 16)], tp_hbm.at[pl.ds(999936 // 4, 16)]
        )


B_PER_W = TOTAL // NW
CHUNK = 832
N_CHUNKS = B_PER_W // CHUNK
NBUF = 4


@functools.partial(
    pl.kernel,
    out_type=jax.ShapeDtypeStruct((TOTAL, EMBED_DIM), jnp.float32),
    mesh=_mesh,
    scratch_types=[
        pltpu.VMEM((B_PER_W,), jnp.int32),
        pltpu.VMEM((NBUF, CHUNK, EMBED_DIM), jnp.float32),
        [pltpu.SemaphoreType.DMA] * NBUF,
        [pltpu.SemaphoreType.DMA] * NBUF,
    ],
    compiler_params=pltpu.CompilerParams(use_tc_tiling_on_sc=False),
)
def _embed_gather(idx_hbm, table_hbm, out_hbm, idx_v, rows_v, gsems, ssems):
    wid = lax.axis_index("s") * NC + lax.axis_index("c")
    base = wid * B_PER_W
    pltpu.sync_copy(idx_hbm.at[pl.ds(base, B_PER_W)], idx_v)

    def start_gather(c):
        b = c % NBUF
        return pltpu.async_copy(
            table_hbm.at[idx_v.at[pl.ds(c * CHUNK, CHUNK)]],
            rows_v.at[b],
            gsems[b],
        )

    def start_store(c):
        b = c % NBUF
        return pltpu.async_copy(
            rows_v.at[b],
            out_hbm.at[pl.ds(base + c * CHUNK, CHUNK)],
            ssems[b],
        )

    gathers = [None] * N_CHUNKS
    stores = [None] * N_CHUNKS
    for c in range(min(NBUF - 1, N_CHUNKS)):
        gathers[c] = start_gather(c)
    for c in range(N_CHUNKS):
        if c > 0:
            stores[c - 1].wait()
        g = c + NBUF - 1
        if g < N_CHUNKS:
            gathers[g] = start_gather(g)
        gathers[c].wait()
        stores[c] = start_store(c)
    stores[N_CHUNKS - 1].wait()


def kernel(x, table):
    flat = x.reshape(TOTAL).astype(jnp.int32)
    tt = table.T
    tail16 = lax.slice(table, (999936, 0), (VOCAB, EMBED_DIM)).reshape(16, 128)
    tp = _table_transpose(tt, tail16)
    tlin = tp.reshape(VOCAB, EMBED_DIM)
    out = _embed_gather(flat, tlin)
    return out.reshape(BATCH, NUM_FIELDS, EMBED_DIM)


# speedup vs baseline: 1.2158x; 1.1563x over previous
import functools

import jax
import jax.numpy as jnp
from jax import lax
from jax.experimental import pallas as pl
from jax.experimental.pallas import tpu as pltpu
from jax.experimental.pallas import tpu_sc as plsc

BATCH = 16384
NUM_FIELDS = 26
EMBED_DIM = 32
VOCAB = 1000000
TOTAL = BATCH * NUM_FIELDS
NC = 2
NS = 16
NW = NC * NS

_mesh = plsc.VectorSubcoreMesh(core_axis_name="c", subcore_axis_name="s")


V_PER_W = 31232
BLK = 1024
N_FULL = V_PER_W // BLK
V_MAIN = NW * V_PER_W
V_REM = 999936 - V_MAIN
V_TAIL = VOCAB - 999936


@functools.partial(
    pl.kernel,
    out_type=jax.ShapeDtypeStruct((VOCAB // 4, 128), jnp.float32),
    mesh=_mesh,
    scratch_types=[
        pltpu.VMEM((2, 32, BLK), jnp.float32),
        pltpu.VMEM((BLK // 4, 128), jnp.float32),
        [pltpu.SemaphoreType.DMA] * 2,
    ],
    compiler_params=pltpu.CompilerParams(
        use_tc_tiling_on_sc=True, needs_layout_passes=False
    ),
)
def _table_transpose(tt_hbm, tail_hbm, tp_hbm, src, dst, gsems):
    wid = lax.axis_index("s") * NC + lax.axis_index("c")
    v0w = wid * V_PER_W

    def issue_reads(v0, width, buf):
        v0 = pl.multiple_of(v0, 128)
        return [
            pltpu.async_copy(
                tt_hbm.at[pl.ds(8 * tr, 8), pl.ds(v0, width)],
                src.at[buf, pl.ds(8 * tr, 8), pl.ds(0, width)],
                gsems[buf],
            )
            for tr in range(4)
        ]

    iota = lax.iota(jnp.int32, 16)
    row_pat = iota // 4
    lane_pat = (iota % 4) * 32

    def transpose_block(buf, width):
        src_b = src.at[buf]

        def body(k, _):
            idx_r = row_pat + 4 * k
            vregs = [src_b[d, pl.ds(16 * k, 16)] for d in range(32)]
            for d in range(32):
                plsc.store_scatter(dst, [idx_r, lane_pat + d], vregs[d])
            return 0

        lax.fori_loop(0, width // 16, body, 0)

    widths = [BLK] * N_FULL + [V_PER_W - N_FULL * BLK]
    v0s = [v0w + k * BLK for k in range(len(widths))]

    reads = [issue_reads(v0s[0], widths[0], 0)]
    for k in range(len(widths)):
        if k + 1 < len(widths):
            reads.append(issue_reads(v0s[k + 1], widths[k + 1], (k + 1) % 2))
        for r_ in reads[k]:
            r_.wait()
        transpose_block(k % 2, widths[k])
        pltpu.sync_copy(
            dst.at[pl.ds(0, widths[k] // 4)],
            tp_hbm.at[pl.ds(pl.multiple_of(v0s[k] // 4, 8), widths[k] // 4)],
        )

    @pl.when(wid == 0)
    def _():
        rem = issue_reads(V_MAIN, V_REM, 0)
        for r_ in rem:
            r_.wait()
        transpose_block(0, V_REM)
        pltpu.sync_copy(
            dst.at[pl.ds(0, V_REM // 4)],
            tp_hbm.at[pl.ds(V_MAIN // 4, V_REM // 4)],
        )

    @pl.when(wid == 1)
    def _():
        pltpu.sync_copy(tail_hbm, dst.at[pl.ds(0, 16)])
        pltpu.sync_copy(
            dst.at[pl.ds(0, 16)], tp_hbm.at[pl.ds(999936 // 4, 16)]
        )


B_PER_W = TOTAL // NW
CHUNK = 832
N_CHUNKS = B_PER_W // CHUNK
NBUF = 4


@functools.partial(
    pl.kernel,
    out_type=jax.ShapeDtypeStruct((TOTAL, EMBED_DIM), jnp.float32),
    mesh=_mesh,
    scratch_types=[
        pltpu.VMEM((B_PER_W,), jnp.int32),
        pltpu.VMEM((NBUF, CHUNK, EMBED_DIM), jnp.float32),
        [pltpu.SemaphoreType.DMA] * NBUF,
        [pltpu.SemaphoreType.DMA] * NBUF,
    ],
    compiler_params=pltpu.CompilerParams(use_tc_tiling_on_sc=False),
)
def _embed_gather(idx_hbm, table_hbm, out_hbm, idx_v, rows_v, gsems, ssems):
    wid = lax.axis_index("s") * NC + lax.axis_index("c")
    base = wid * B_PER_W
    pltpu.sync_copy(idx_hbm.at[pl.ds(base, B_PER_W)], idx_v)

    def start_gather(c):
        b = c % NBUF
        return pltpu.async_copy(
            table_hbm.at[idx_v.at[pl.ds(c * CHUNK, CHUNK)]],
            rows_v.at[b],
            gsems[b],
        )

    def start_store(c):
        b = c % NBUF
        return pltpu.async_copy(
            rows_v.at[b],
            out_hbm.at[pl.ds(base + c * CHUNK, CHUNK)],
            ssems[b],
        )

    gathers = [None] * N_CHUNKS
    stores = [None] * N_CHUNKS
    for c in range(min(NBUF - 1, N_CHUNKS)):
        gathers[c] = start_gather(c)
    for c in range(N_CHUNKS):
        if c > 0:
            stores[c - 1].wait()
        g = c + NBUF - 1
        if g < N_CHUNKS:
            gathers[g] = start_gather(g)
        gathers[c].wait()
        stores[c] = start_store(c)
    stores[N_CHUNKS - 1].wait()


def kernel(x, table):
    flat = x.reshape(TOTAL).astype(jnp.int32)
    tt = table.T
    tail16 = lax.slice(table, (999936, 0), (VOCAB, EMBED_DIM)).reshape(16, 128)
    tp = _table_transpose(tt, tail16)
    tlin = tp.reshape(VOCAB, EMBED_DIM)
    out = _embed_gather(flat, tlin)
    return out.reshape(BATCH, NUM_FIELDS, EMBED_DIM)


# speedup vs baseline: 1.3059x; 1.0740x over previous
import functools

import jax
import jax.numpy as jnp
from jax import lax
from jax.experimental import pallas as pl
from jax.experimental.pallas import tpu as pltpu
from jax.experimental.pallas import tpu_sc as plsc

BATCH = 16384
NUM_FIELDS = 26
EMBED_DIM = 32
VOCAB = 1000000
TOTAL = BATCH * NUM_FIELDS
NC = 2
NS = 16
NW = NC * NS

_mesh = plsc.VectorSubcoreMesh(core_axis_name="c", subcore_axis_name="s")


C_PER_W = 244
N_MAIN_CH = NW * C_PER_W
V_MAIN = N_MAIN_CH * 128


@functools.partial(
    pl.kernel,
    out_type=jax.ShapeDtypeStruct((VOCAB // 4, 128), jnp.float32),
    mesh=_mesh,
    scratch_types=[
        pltpu.VMEM((2, 32, 128), jnp.float32),
        pltpu.VMEM((2, 32, 128), jnp.float32),
        pltpu.SemaphoreType.DMA,
        pltpu.SemaphoreType.DMA,
    ],
    compiler_params=pltpu.CompilerParams(
        use_tc_tiling_on_sc=True, needs_layout_passes=False
    ),
)
def _table_transpose(tt_hbm, tail_hbm, tp_hbm, src, dst, rsem, wsem):
    wid = lax.axis_index("s") * NC + lax.axis_index("c")
    c0 = wid * C_PER_W

    iota = lax.iota(jnp.int32, 16)
    row_pat = iota // 4
    lane_pat = (iota % 4) * 32

    def rd(ci, b):
        return pltpu.async_copy(
            tt_hbm.at[:, pl.ds(pl.multiple_of(ci * 128, 128), 128)],
            src.at[b],
            rsem,
        )

    def wr(ci, b):
        return pltpu.async_copy(
            dst.at[b],
            tp_hbm.at[pl.ds(pl.multiple_of(ci * 32, 32), 32)],
            wsem,
        )

    def transpose_chunk(src_b, dst_b):
        for k in range(8):
            idx_r = row_pat + 4 * k
            vregs = [src_b[d, pl.ds(16 * k, 16)] for d in range(32)]
            for d in range(32):
                plsc.store_scatter(dst_b, [idx_r, lane_pat + d], vregs[d])

    rd(c0, 0)

    def body(i, _):
        b = lax.rem(i, 2)

        @pl.when(i + 1 < C_PER_W)
        def _():
            rd(c0 + i + 1, 1 - b)

        pltpu.make_async_copy(
            tt_hbm.at[:, pl.ds(0, 128)], src.at[b], rsem
        ).wait()

        @pl.when(i >= 2)
        def _():
            pltpu.make_async_copy(
                dst.at[b], tp_hbm.at[pl.ds(0, 32)], wsem
            ).wait()

        transpose_chunk(src.at[b], dst.at[b])
        wr(c0 + i, b)
        return 0

    lax.fori_loop(0, C_PER_W, body, 0)
    for j in range(2):
        pltpu.make_async_copy(
            dst.at[j], tp_hbm.at[pl.ds(0, 32)], wsem
        ).wait()

    @pl.when(wid < 4)
    def _():
        ci = N_MAIN_CH + wid
        rd(ci, 0).wait()
        transpose_chunk(src.at[0], dst.at[0])
        wr(ci, 0).wait()

    @pl.when(wid == 4)
    def _():
        pltpu.sync_copy(tail_hbm, src.at[0, pl.ds(0, 16)])
        pltpu.sync_copy(
            src.at[0, pl.ds(0, 16)], tp_hbm.at[pl.ds(999936 // 4, 16)]
        )


B_PER_W = TOTAL // NW
CHUNK = 832
N_CHUNKS = B_PER_W // CHUNK
NBUF = 4


@functools.partial(
    pl.kernel,
    out_type=jax.ShapeDtypeStruct((TOTAL, EMBED_DIM), jnp.float32),
    mesh=_mesh,
    scratch_types=[
        pltpu.VMEM((B_PER_W,), jnp.int32),
        pltpu.VMEM((NBUF, CHUNK, EMBED_DIM), jnp.float32),
        [pltpu.SemaphoreType.DMA] * NBUF,
        [pltpu.SemaphoreType.DMA] * NBUF,
    ],
    compiler_params=pltpu.CompilerParams(use_tc_tiling_on_sc=False),
)
def _embed_gather(idx_hbm, table_hbm, out_hbm, idx_v, rows_v, gsems, ssems):
    wid = lax.axis_index("s") * NC + lax.axis_index("c")
    base = wid * B_PER_W
    pltpu.sync_copy(idx_hbm.at[pl.ds(base, B_PER_W)], idx_v)

    def start_gather(c):
        b = c % NBUF
        return pltpu.async_copy(
            table_hbm.at[idx_v.at[pl.ds(c * CHUNK, CHUNK)]],
            rows_v.at[b],
            gsems[b],
        )

    def start_store(c):
        b = c % NBUF
        return pltpu.async_copy(
            rows_v.at[b],
            out_hbm.at[pl.ds(base + c * CHUNK, CHUNK)],
            ssems[b],
        )

    gathers = [None] * N_CHUNKS
    stores = [None] * N_CHUNKS
    for c in range(min(NBUF - 1, N_CHUNKS)):
        gathers[c] = start_gather(c)
    for c in range(N_CHUNKS):
        if c > 0:
            stores[c - 1].wait()
        g = c + NBUF - 1
        if g < N_CHUNKS:
            gathers[g] = start_gather(g)
        gathers[c].wait()
        stores[c] = start_store(c)
    stores[N_CHUNKS - 1].wait()


def kernel(x, table):
    flat = x.reshape(TOTAL).astype(jnp.int32)
    tt = table.T
    tail16 = lax.slice(table, (999936, 0), (VOCAB, EMBED_DIM)).reshape(16, 128)
    tp = _table_transpose(tt, tail16)
    tlin = tp.reshape(VOCAB, EMBED_DIM)
    out = _embed_gather(flat, tlin)
    return out.reshape(BATCH, NUM_FIELDS, EMBED_DIM)
